# baseline (speedup 1.0000x reference)
import jax
import jax.numpy as jnp
from jax.experimental import pallas as pl
from jax.experimental.pallas import tpu as pltpu


def _body(sb_hbm, x_ref, o_ref, sb_v, sem):
    @pl.when(pl.program_id(0) == 0)
    def _():
        cp = pltpu.make_async_copy(sb_hbm, sb_v, sem)
        cp.start()
        cp.wait()

    scale = sb_v[:, 0:1]
    bias = sb_v[:, 1:2]
    o_ref[...] = x_ref[...] * scale + bias


def kernel(inputs, mask, alpha, beta):
    B, C = inputs.shape
    xt = inputs.T
    col = jnp.arange(2, dtype=jnp.int32)[None, :]
    sb = jnp.where(
        mask[:, None],
        jnp.where(col == 0, alpha[0], beta[0]),
        jnp.where(col == 0, 1.0, 0.0),
    ).astype(jnp.float32)
    blk = 2048
    out_t = pl.pallas_call(
        _body,
        grid=(B // blk,),
        in_specs=[
            pl.BlockSpec(memory_space=pl.ANY),
            pl.BlockSpec((C, blk), lambda i: (0, i)),
        ],
        out_specs=pl.BlockSpec((C, blk), lambda i: (0, i)),
        out_shape=jax.ShapeDtypeStruct((C, B), jnp.float32),
        scratch_shapes=[
            pltpu.VMEM((C, 2), jnp.float32),
            pltpu.SemaphoreType.DMA,
        ],
    )(sb, xt)
    return out_t.T


# speedup vs baseline: 1.0420x; 1.0420x over previous
import jax
import jax.numpy as jnp
from jax.experimental import pallas as pl
from jax.experimental.pallas import tpu as pltpu


def _body(m_ref, x_ref, o_ref):
    C = m_ref.shape[0] - 2
    m = m_ref[0:C, :]
    a = m_ref[C:C + 1, :]
    b = m_ref[C + 1:C + 2, :]
    scale = 1.0 + m * (a - 1.0)
    bias = m * b
    o_ref[...] = x_ref[...] * scale + bias


def kernel(inputs, mask, alpha, beta):
    B, C = inputs.shape
    xt = inputs.T
    mcol = jnp.concatenate(
        [mask[:, None].astype(jnp.float32), alpha[:, None], beta[:, None]],
        axis=0,
    )
    blk = 2048
    out_t = pl.pallas_call(
        _body,
        grid=(B // blk,),
        in_specs=[
            pl.BlockSpec((C + 2, 1), lambda i: (0, 0)),
            pl.BlockSpec((C, blk), lambda i: (0, i)),
        ],
        out_specs=pl.BlockSpec((C, blk), lambda i: (0, i)),
        out_shape=jax.ShapeDtypeStruct((C, B), jnp.float32),
    )(mcol, xt)
    return out_t.T


# speedup vs baseline: 1.0464x; 1.0043x over previous
import jax
import jax.numpy as jnp
from jax import lax
from jax.experimental import pallas as pl
from jax.experimental.pallas import tpu as pltpu


def _body(sb_ref, x_ref, o_ref):
    scale = sb_ref[:, 0:1]
    bias = sb_ref[:, 1:2]
    o_ref[...] = x_ref[...] * scale + bias


def kernel(inputs, mask, alpha, beta):
    B, C = inputs.shape
    xt = inputs.T
    col = lax.broadcasted_iota(jnp.int32, (C, 2), 1)
    sb = jnp.where(
        mask[:, None],
        jnp.where(col == 0, alpha[0], beta[0]),
        jnp.where(col == 0, 1.0, 0.0),
    ).astype(jnp.float32)
    blk = 2048
    out_t = pl.pallas_call(
        _body,
        grid=(B // blk,),
        in_specs=[
            pl.BlockSpec((C, 2), lambda i: (0, 0)),
            pl.BlockSpec((C, blk), lambda i: (0, i)),
        ],
        out_specs=pl.BlockSpec((C, blk), lambda i: (0, i)),
        out_shape=jax.ShapeDtypeStruct((C, B), jnp.float32),
    )(sb, xt)
    return out_t.T
